# Initial kernel scaffold; baseline (speedup 1.0000x reference)
import jax
import jax.numpy as jnp
from jax.experimental import pallas as pl


def kernel(boxes, scores):
    raise NotImplementedError("write your pallas kernel here")


# speedup vs baseline: 18.4288x; 18.4288x over previous
import functools

import jax
import jax.numpy as jnp
from jax.experimental import pallas as pl
from jax.experimental.pallas import tpu as pltpu

_N = 20000
_MAX_OUT = 100
_IOU_T = 0.5
_LANES = 128
_ROWS = 160
_NPAD = _ROWS * _LANES
_NEG = float("-inf")


def _nms_body(x1_ref, y1_ref, x2_ref, y2_ref, sc_ref, out_ref, ms_ref):
    ms_ref[...] = sc_ref[...]

    coli = jax.lax.broadcasted_iota(jnp.int32, (1, _LANES), 1)
    rowi = jax.lax.broadcasted_iota(jnp.int32, (_ROWS, _LANES), 0)
    cols = jax.lax.broadcasted_iota(jnp.int32, (_ROWS, _LANES), 1)
    lin = rowi * _LANES + cols

    def extract_lane(row, c):
        return jnp.sum(jnp.where(coli == c, row, 0.0))

    def cond(carry):
        nk, alive = carry[0], carry[1]
        return jnp.logical_and(nk < _MAX_OUT, alive)

    def body(carry):
        nk, alive, kx1, ky1, kx2, ky2, karea, ks = carry
        ms = ms_ref[...]
        m = jnp.max(ms)
        alive = m > _NEG
        cand = jnp.min(jnp.where(ms == m, lin, jnp.int32(2**30)))
        r = cand // _LANES
        c = cand - r * _LANES
        cx1 = extract_lane(x1_ref[pl.ds(r, 1), :], c)
        cy1 = extract_lane(y1_ref[pl.ds(r, 1), :], c)
        cx2 = extract_lane(x2_ref[pl.ds(r, 1), :], c)
        cy2 = extract_lane(y2_ref[pl.ds(r, 1), :], c)
        carea = jnp.maximum(cx2 - cx1, 0.0) * jnp.maximum(cy2 - cy1, 0.0)
        xx1 = jnp.maximum(kx1, cx1)
        yy1 = jnp.maximum(ky1, cy1)
        xx2 = jnp.minimum(kx2, cx2)
        yy2 = jnp.minimum(ky2, cy2)
        inter = jnp.maximum(xx2 - xx1, 0.0) * jnp.maximum(yy2 - yy1, 0.0)
        iou = inter / (karea + carea - inter + 1e-9)
        suppressed = jnp.any(iou > _IOU_T)
        keep = jnp.logical_and(alive, jnp.logical_not(suppressed))
        app = jnp.logical_and(keep, coli == nk)
        kx1 = jnp.where(app, cx1, kx1)
        ky1 = jnp.where(app, cy1, ky1)
        kx2 = jnp.where(app, cx2, kx2)
        ky2 = jnp.where(app, cy2, ky2)
        karea = jnp.where(app, carea, karea)
        ks = jnp.where(app, m, ks)
        nk = nk + keep.astype(jnp.int32)
        msrow = ms_ref[pl.ds(r, 1), :]
        ms_ref[pl.ds(r, 1), :] = jnp.where(coli == c, _NEG, msrow)
        return (nk, alive, kx1, ky1, kx2, ky2, karea, ks)

    zro = jnp.zeros((1, _LANES), jnp.float32)
    init = (jnp.int32(0), jnp.bool_(True), zro, zro, zro, zro, zro, zro)
    nk, alive, kx1, ky1, kx2, ky2, karea, ks = jax.lax.while_loop(
        cond, body, init)
    out_ref[0:1, :] = kx1
    out_ref[1:2, :] = ky1
    out_ref[2:3, :] = kx2
    out_ref[3:4, :] = ky2
    out_ref[4:5, :] = ks
    out_ref[5:8, :] = jnp.zeros((3, _LANES), jnp.float32)


@jax.jit
def kernel(boxes, scores):
    pad = _NPAD - _N
    x1 = jnp.pad(boxes[:, 0], (0, pad)).reshape(_ROWS, _LANES)
    y1 = jnp.pad(boxes[:, 1], (0, pad)).reshape(_ROWS, _LANES)
    x2 = jnp.pad(boxes[:, 2], (0, pad)).reshape(_ROWS, _LANES)
    y2 = jnp.pad(boxes[:, 3], (0, pad)).reshape(_ROWS, _LANES)
    sc = jnp.pad(scores, (0, pad), constant_values=_NEG).reshape(
        _ROWS, _LANES)
    res = pl.pallas_call(
        _nms_body,
        out_shape=jax.ShapeDtypeStruct((8, _LANES), jnp.float32),
        scratch_shapes=[pltpu.VMEM((_ROWS, _LANES), jnp.float32)],
    )(x1, y1, x2, y2, sc)
    return res[:5, :_MAX_OUT].T
